# baseline (speedup 1.0000x reference)
import functools

import jax
import jax.numpy as jnp
from jax import lax
from jax.experimental import pallas as pl
from jax.experimental.pallas import tpu as pltpu
from jax.experimental.pallas import tpu_sc as plsc

VOCAB = 1000000
D = 64
B = 819200
NC = 2
NS = 16
NW = NC * NS
C = 128
K = 8
ROWS_PER_STEP = K * C
B_PER_W = B // NW
STEPS = B_PER_W // ROWS_PER_STEP


def _sc_gather(idx2d, table):
    mesh = plsc.VectorSubcoreMesh(core_axis_name="c", subcore_axis_name="s")

    @functools.partial(
        pl.kernel,
        mesh=mesh,
        compiler_params=pltpu.CompilerParams(use_tc_tiling_on_sc=False),
        out_type=jax.ShapeDtypeStruct((B, D), jnp.float32),
        scratch_types=[
            pltpu.VMEM((K, C), jnp.int32),
            pltpu.VMEM((ROWS_PER_STEP, D), jnp.float32),
            pltpu.SemaphoreType.DMA,
        ],
    )
    def k(idx_hbm, table_hbm, out_hbm, idx_v, rows_v, sem):
        wid = lax.axis_index("s") * NC + lax.axis_index("c")
        blk0 = wid * (B_PER_W // C)

        def body(g, carry):
            blk = blk0 + g * K
            pltpu.sync_copy(idx_hbm.at[pl.ds(blk, K)], idx_v)
            copies = [
                pltpu.async_copy(
                    table_hbm.at[idx_v.at[j]],
                    rows_v.at[pl.ds(j * C, C)],
                    sem,
                )
                for j in range(K)
            ]
            for cp in copies:
                cp.wait()
            pltpu.sync_copy(rows_v, out_hbm.at[pl.ds(blk * C, ROWS_PER_STEP)])
            return carry

        lax.fori_loop(0, STEPS, body, 0)

    return k(idx2d, table)


def kernel(indices, table):
    idx2d = indices.astype(jnp.int32).reshape(B // C, C)
    return _sc_gather(idx2d, table)


# speedup vs baseline: 1.0180x; 1.0180x over previous
import functools

import jax
import jax.numpy as jnp
from jax import lax
from jax.experimental import pallas as pl
from jax.experimental.pallas import tpu as pltpu
from jax.experimental.pallas import tpu_sc as plsc

VOCAB = 1000000
D = 64
B = 819200
NC = 2
NS = 16
NW = NC * NS
C = 128
K = 2
RPS = K * C
B_PER_W = B // NW
NBLK = B_PER_W // C
STEPS = B_PER_W // RPS
NBUF = 4


def _sc_gather(idx2d, table):
    mesh = plsc.VectorSubcoreMesh(core_axis_name="c", subcore_axis_name="s")

    @functools.partial(
        pl.kernel,
        mesh=mesh,
        compiler_params=pltpu.CompilerParams(use_tc_tiling_on_sc=False),
        out_type=jax.ShapeDtypeStruct((B, D), jnp.float32),
        scratch_types=[
            pltpu.VMEM((NBLK, C), jnp.int32),
            [pltpu.VMEM((RPS, D), jnp.float32)] * NBUF,
            [pltpu.SemaphoreType.DMA] * NBUF,
            [pltpu.SemaphoreType.DMA] * NBUF,
        ],
    )
    def k(idx_hbm, table_hbm, out_hbm, idx_all, rows, gsem, wsem):
        wid = lax.axis_index("s") * NC + lax.axis_index("c")
        blk0 = wid * NBLK
        pltpu.sync_copy(idx_hbm.at[pl.ds(blk0, NBLK)], idx_all)

        def fire_gather(i, b):
            for j in range(K):
                pltpu.async_copy(
                    table_hbm.at[idx_all.at[i * K + j]],
                    rows[b].at[pl.ds(j * C, C)],
                    gsem[b],
                )

        def drain_gather(b):
            pltpu.make_async_copy(
                table_hbm.at[pl.ds(0, RPS)], rows[b], gsem[b]
            ).wait()

        def fire_write(i, b):
            pltpu.async_copy(
                rows[b], out_hbm.at[pl.ds((blk0 + i * K) * C, RPS)], wsem[b]
            )

        def drain_write(b):
            pltpu.make_async_copy(
                rows[b], out_hbm.at[pl.ds(blk0 * C, RPS)], wsem[b]
            ).wait()

        fire_gather(0, 0)
        fire_gather(1, 1)
        drain_gather(0)
        fire_write(0, 0)
        fire_gather(2, 2)
        drain_gather(1)
        fire_write(1, 1)
        fire_gather(3, 3)

        def body(t, carry):
            base = 2 + t * 4
            for u in range(4):
                i = base + u
                b = (2 + u) % NBUF
                drain_gather(b)
                fire_write(i, b)
                drain_write(u % NBUF)
                fire_gather(i + 2, u % NBUF)
            return carry

        lax.fori_loop(0, (STEPS - 4) // 4, body, 0)

        drain_gather(2)
        fire_write(STEPS - 2, 2)
        drain_write(0)
        drain_gather(3)
        fire_write(STEPS - 1, 3)
        drain_write(1)
        drain_write(2)
        drain_write(3)

    return k(idx2d, table)


def kernel(indices, table):
    idx2d = indices.astype(jnp.int32).reshape(B // C, C)
    return _sc_gather(idx2d, table)
